# baseline (speedup 1.0000x reference)
import functools

import jax
import jax.numpy as jnp
from jax import lax
from jax.experimental import pallas as pl
from jax.experimental.pallas import tpu as pltpu
from jax.experimental.pallas import tpu_sc as plsc

NUM_CLASSES = 1000
N = 16384
OUT_DTYPE = jnp.result_type(jnp.int64)

NC, NS = 2, 16
NW = NC * NS
ROWS_W = N // NW
CHUNK = 16
NCHUNK = ROWS_W // CHUNK
NBUF = 2


def _zero_buffer(buf, b, zeros, iota16):
    for r in range(CHUNK):
        for c in range(62):
            buf[b, r, pl.ds(c * 16, 16)] = zeros
        plsc.store_scatter(
            buf.at[b], [jnp.full((16,), r, jnp.int32), 984 + iota16], zeros
        )


def _sc_onehot(idx_hbm, out_hbm, idxbuf, buf, sem0, sem1):
    wid = lax.axis_index("s") * NC + lax.axis_index("c")
    base = wid * ROWS_W
    sems = (sem0, sem1)

    pltpu.sync_copy(idx_hbm.at[pl.ds(base, ROWS_W)], idxbuf)

    iota16 = lax.iota(jnp.int32, 16)
    zeros = jnp.zeros((16,), jnp.int32)
    ones = jnp.ones((16,), OUT_DTYPE)

    for b in range(NBUF):
        _zero_buffer(buf, b, zeros, iota16)

    for g in range(NCHUNK):
        b = g % NBUF
        if g >= NBUF:
            pltpu.make_async_copy(
                buf.at[b], out_hbm.at[pl.ds(base + (g - NBUF) * CHUNK, CHUNK), :],
                sems[b],
            ).wait()
            prev_cols = idxbuf[pl.ds((g - NBUF) * CHUNK, 16)]
            plsc.store_scatter(buf.at[b], [iota16, prev_cols], zeros)
        cols = idxbuf[pl.ds(g * CHUNK, 16)]
        plsc.store_scatter(buf.at[b], [iota16, cols], ones)
        pltpu.make_async_copy(
            buf.at[b], out_hbm.at[pl.ds(base + g * CHUNK, CHUNK), :], sems[b]
        ).start()

    for g in range(NCHUNK - NBUF, NCHUNK):
        b = g % NBUF
        pltpu.make_async_copy(
            buf.at[b], out_hbm.at[pl.ds(base + g * CHUNK, CHUNK), :], sems[b]
        ).wait()


def kernel(input):
    mesh = plsc.VectorSubcoreMesh(core_axis_name="c", subcore_axis_name="s")
    sc_call = functools.partial(
        pl.kernel,
        mesh=mesh,
        out_type=jax.ShapeDtypeStruct((N, NUM_CLASSES), OUT_DTYPE),
        scratch_types=[
            pltpu.VMEM((ROWS_W,), jnp.int32),
            pltpu.VMEM((NBUF, CHUNK, NUM_CLASSES), OUT_DTYPE),
            pltpu.SemaphoreType.DMA,
            pltpu.SemaphoreType.DMA,
        ],
        compiler_params=pltpu.CompilerParams(use_tc_tiling_on_sc=False, needs_layout_passes=False),
    )(_sc_onehot)
    return sc_call(input.astype(jnp.int32))


# speedup vs baseline: 1.5946x; 1.5946x over previous
import functools

import jax
import jax.numpy as jnp
from jax import lax
from jax.experimental import pallas as pl
from jax.experimental.pallas import tpu as pltpu
from jax.experimental.pallas import tpu_sc as plsc

NUM_CLASSES = 1000
N = 16384
OUT_DTYPE = jnp.result_type(jnp.int64)

NC, NS = 2, 16
NW = NC * NS
ROWS_W = N // NW
CHUNK = 16
NCHUNK = ROWS_W // CHUNK
NBUF = 2


def _zero_buffer(buf, b, zeros, iota16):
    for r in range(CHUNK):
        for c in range(62):
            buf[b, r, pl.ds(c * 16, 16)] = zeros
        plsc.store_scatter(
            buf.at[b], [jnp.full((16,), r, jnp.int32), 984 + iota16], zeros
        )


def _sc_onehot(idx_hbm, out_hbm, idxbuf, buf, sem0, sem1):
    wid = lax.axis_index("s") * NC + lax.axis_index("c")
    base = wid * ROWS_W
    sems = (sem0, sem1)

    pltpu.sync_copy(idx_hbm.at[pl.ds(base, ROWS_W)], idxbuf)

    iota16 = lax.iota(jnp.int32, 16)
    zeros = jnp.zeros((16,), jnp.int32)
    ones = jnp.ones((16,), OUT_DTYPE)

    for b in range(NBUF):
        _zero_buffer(buf, b, zeros, iota16)

    for g in range(NCHUNK):
        b = g % NBUF
        if g >= NBUF:
            pltpu.make_async_copy(
                buf.at[b], out_hbm.at[pl.ds(base + (g - NBUF) * CHUNK, CHUNK), :],
                sems[b],
            ).wait()
            prev_cols = idxbuf[pl.ds((g - NBUF) * CHUNK, 16)]
            plsc.store_scatter(buf.at[b], [iota16, prev_cols], zeros)
        cols = idxbuf[pl.ds(g * CHUNK, 16)]
        plsc.store_scatter(buf.at[b], [iota16, cols], ones)
        pltpu.make_async_copy(
            buf.at[b], out_hbm.at[pl.ds(base + g * CHUNK, CHUNK), :], sems[b]
        ).start()

    for g in range(NCHUNK - NBUF, NCHUNK):
        b = g % NBUF
        pltpu.make_async_copy(
            buf.at[b], out_hbm.at[pl.ds(base + g * CHUNK, CHUNK), :], sems[b]
        ).wait()


def kernel(input):
    mesh = plsc.VectorSubcoreMesh(core_axis_name="c", subcore_axis_name="s")
    sc_call = functools.partial(
        pl.kernel,
        mesh=mesh,
        out_type=jax.ShapeDtypeStruct((N, NUM_CLASSES), OUT_DTYPE),
        scratch_types=[
            pltpu.VMEM((ROWS_W,), jnp.int32),
            pltpu.VMEM((NBUF, CHUNK, NUM_CLASSES), OUT_DTYPE),
            pltpu.SemaphoreType.DMA,
            pltpu.SemaphoreType.DMA,
        ],
        compiler_params=pltpu.CompilerParams(use_tc_tiling_on_sc=True, needs_layout_passes=False),
    )(_sc_onehot)
    return sc_call(input.astype(jnp.int32))


# speedup vs baseline: 1.9351x; 1.2135x over previous
import jax
import jax.numpy as jnp
from jax.experimental import pallas as pl
from jax.experimental.pallas import tpu as pltpu

NUM_CLASSES = 1000
N = 16384
OUT_DTYPE = jnp.result_type(jnp.int64)
ROWS = 512
GRID = N // ROWS


def _onehot_block(idx_ref, out_ref):
    classes = jax.lax.broadcasted_iota(jnp.int32, (ROWS, NUM_CLASSES), 1)
    idx = idx_ref[...]
    out_ref[...] = (classes == idx[:, None]).astype(OUT_DTYPE)


def kernel(input):
    return pl.pallas_call(
        _onehot_block,
        grid=(GRID,),
        in_specs=[pl.BlockSpec((ROWS,), lambda i: (i,))],
        out_specs=pl.BlockSpec((ROWS, NUM_CLASSES), lambda i: (i, 0)),
        out_shape=jax.ShapeDtypeStruct((N, NUM_CLASSES), OUT_DTYPE),
        compiler_params=pltpu.CompilerParams(
            dimension_semantics=("arbitrary",),
        ),
    )(input.astype(jnp.int32))


# speedup vs baseline: 1.9428x; 1.0040x over previous
import jax
import jax.numpy as jnp
from jax.experimental import pallas as pl
from jax.experimental.pallas import tpu as pltpu

NUM_CLASSES = 1000
N = 16384
OUT_DTYPE = jnp.result_type(jnp.int64)
ROWS = 512
GRID = N // ROWS


def _onehot_block(idx_ref, out_ref):
    classes = jax.lax.broadcasted_iota(jnp.int32, (ROWS, NUM_CLASSES), 1)
    idx = idx_ref[...]
    out_ref[...] = (classes == idx[:, None]).astype(OUT_DTYPE)


def kernel(input):
    return pl.pallas_call(
        _onehot_block,
        grid=(GRID,),
        in_specs=[pl.BlockSpec((ROWS,), lambda i: (i,))],
        out_specs=pl.BlockSpec((ROWS, NUM_CLASSES), lambda i: (i, 0)),
        out_shape=jax.ShapeDtypeStruct((N, NUM_CLASSES), OUT_DTYPE),
        compiler_params=pltpu.CompilerParams(
            dimension_semantics=("parallel",),
        ),
    )(input.astype(jnp.int32))


# speedup vs baseline: 2.1261x; 1.0944x over previous
import jax
import jax.numpy as jnp
from jax.experimental import pallas as pl
from jax.experimental.pallas import tpu as pltpu

NUM_CLASSES = 1000
N = 16384
OUT_DTYPE = jnp.result_type(jnp.int64)
ROWS = 2048
GRID = N // ROWS


def _onehot_block(idx_ref, out_ref):
    classes = jax.lax.broadcasted_iota(jnp.int32, (ROWS, NUM_CLASSES), 1)
    idx = idx_ref[...]
    out_ref[...] = (classes == idx[:, None]).astype(OUT_DTYPE)


def kernel(input):
    return pl.pallas_call(
        _onehot_block,
        grid=(GRID,),
        in_specs=[pl.BlockSpec((ROWS,), lambda i: (i,))],
        out_specs=pl.BlockSpec((ROWS, NUM_CLASSES), lambda i: (i, 0)),
        out_shape=jax.ShapeDtypeStruct((N, NUM_CLASSES), OUT_DTYPE),
        compiler_params=pltpu.CompilerParams(
            dimension_semantics=("parallel",),
        ),
    )(input.astype(jnp.int32))
